# baseline (speedup 1.0000x reference)
import functools

import jax
import jax.numpy as jnp
from jax import lax
from jax.experimental import pallas as pl
from jax.experimental.pallas import tpu as pltpu
from jax.experimental.pallas import tpu_sc as plsc

_F32 = jnp.float32
_HI = lax.Precision.HIGHEST

_NC = 2
_NS = 16


def _dotT(a, b):
    return lax.dot_general(a, b, (((1,), (1,)), ((), ())),
                           preferred_element_type=_F32, precision=_HI)


def _k1_body(obj_ref, ws_ref, bs_ref, wo_ref, bo_ref, hs_ref, ho_ref, om_ref):
    obj = obj_ref[...]
    n = obj.shape[0]
    hs_ref[...] = _dotT(obj, ws_ref[...]) + bs_ref[...]
    ho_ref[...] = _dotT(obj, wo_ref[...]) + bo_ref[...]
    g = _dotT(obj, obj)
    s1 = jnp.sum(obj * obj, axis=1)
    sq = s1[:, None] + s1[None, :] - 2.0 * g
    om = jnp.where(sq < 0.25, 4.0, jnp.where(sq < 1.0, 1.0 / sq, 0.0))
    ii = lax.broadcasted_iota(jnp.int32, (n, n), 0)
    jj = lax.broadcasted_iota(jnp.int32, (n, n), 1)
    om_ref[...] = jnp.where(ii == jj, 0.0, om)


def _k1(obj, Ws, bs2, Wo, bo2):
    n, d = obj.shape
    return pl.pallas_call(
        _k1_body,
        out_shape=[jax.ShapeDtypeStruct((n, d), _F32),
                   jax.ShapeDtypeStruct((n, d), _F32),
                   jax.ShapeDtypeStruct((n, n), _F32)],
    )(obj, Ws, bs2, Wo, bo2)


def _k2(hs, ho, src, dst):
    n, d = hs.shape
    e = src.shape[0]
    nw = _NC * _NS
    e_per_w = e // nw
    chunk = 64
    nchunk = e_per_w // chunk
    mesh = plsc.VectorSubcoreMesh(core_axis_name="c", subcore_axis_name="s",
                                  num_cores=_NC, num_subcores=_NS)

    @functools.partial(
        pl.kernel,
        out_type=[jax.ShapeDtypeStruct((e, d), _F32),
                  jax.ShapeDtypeStruct((e, d), _F32)],
        mesh=mesh,
        scratch_types=[
            pltpu.VMEM((chunk,), jnp.int32),
            pltpu.VMEM((chunk, d), _F32),
            pltpu.SemaphoreType.DMA,
        ],
    )
    def k2(hs_hbm, ho_hbm, src_hbm, dst_hbm, hsg_hbm, hog_hbm,
           idx_v, rows_v, sem):
        wid = lax.axis_index("s") * _NC + lax.axis_index("c")
        for tab, idxarr, out in ((hs_hbm, src_hbm, hsg_hbm),
                                 (ho_hbm, dst_hbm, hog_hbm)):
            for c in range(nchunk):
                base = wid * e_per_w + c * chunk
                pltpu.sync_copy(idxarr.at[pl.ds(base, chunk)], idx_v)
                pltpu.async_copy(tab.at[idx_v], rows_v, sem).wait()
                pltpu.sync_copy(rows_v, out.at[pl.ds(base, chunk)])

    return k2(hs, ho, src, dst)


def _k3_body(hsg_ref, hog_ref, un_ref, ww_ref, bw_ref, flat_ref, out_ref):
    m = hsg_ref[...] * hog_ref[...] * un_ref[...]
    be, p = out_ref.shape[0], ww_ref.shape[0]
    af = _dotT(m, ww_ref[...]) + bw_ref[...]
    flat = flat_ref[0, 0, :]
    af16 = jnp.broadcast_to(af[:, None, :], (be, 16, p)).reshape(be, 16 * p)
    lane = lax.broadcasted_iota(jnp.int32, (be, 16 * p), 1)
    sel = (lane // p) == (flat % 16)[:, None]
    out_ref[...] = jnp.where(sel, af16, 0.0)


def _k3(hsg, hog, union, Ww, bw2, flat3):
    e, d = hsg.shape
    p = Ww.shape[0]
    be = 1024
    grid = (e // be,)
    row_spec = pl.BlockSpec((be, d), lambda i: (i, 0))
    return pl.pallas_call(
        _k3_body,
        grid=grid,
        in_specs=[row_spec, row_spec, row_spec,
                  pl.BlockSpec((p, d), lambda i: (0, 0)),
                  pl.BlockSpec((1, p), lambda i: (0, 0)),
                  pl.BlockSpec((1, 1, be), lambda i: (i, 0, 0))],
        out_specs=pl.BlockSpec((be, 16 * p), lambda i: (i, 0)),
        out_shape=jax.ShapeDtypeStruct((e, 16 * p), _F32),
    )(hsg, hog, union, Ww, bw2, flat3)


def _k4(af128, src, dst, zrows, n):
    e = af128.shape[0]
    npass = 2
    region = n * n // (npass * _NC)
    r16 = region // 16
    ept = e // _NS
    rpt = r16 // _NS
    ng = ept // 128
    mesh = plsc.VectorSubcoreMesh(core_axis_name="c", subcore_axis_name="s",
                                  num_cores=_NC, num_subcores=_NS)

    @functools.partial(
        pl.kernel,
        out_type=jax.ShapeDtypeStruct((n * n // 16, 128), _F32),
        mesh=mesh,
        scratch_types=[
            pltpu.VMEM((128, 128), _F32),
            pltpu.VMEM((ept,), jnp.int32),
            pltpu.VMEM((ept,), jnp.int32),
            pltpu.VMEM((ng, 128), jnp.int32),
            pltpu.VMEM_SHARED((r16 + 1, 128), _F32),
            pltpu.SemaphoreType.DMA,
        ],
    )
    def k4(af_hbm, src_hbm, dst_hbm, z_hbm, out_hbm,
           vals_v, src_v, dst_v, idx_v, acc_sh, sem):
        c = lax.axis_index("c")
        s = lax.axis_index("s")
        ebase = s * ept
        pltpu.sync_copy(src_hbm.at[pl.ds(ebase, ept)], src_v)
        pltpu.sync_copy(dst_hbm.at[pl.ds(ebase, ept)], dst_v)
        for q in range(npass):
            lo = (q * _NC + c) * region
            pltpu.sync_copy(z_hbm, acc_sh.at[pl.ds(s * rpt, rpt)])
            for k in range(ept // 16):
                s16 = src_v[pl.ds(k * 16, 16)]
                d16 = dst_v[pl.ds(k * 16, 16)]
                flat = s16 * n + d16
                inh = (flat >= lo) & (flat < lo + region)
                row = lax.shift_right_arithmetic(flat - lo, 4)
                idx_v[k // 8, pl.ds((k % 8) * 16, 16)] = jnp.where(inh, row, r16)
            plsc.subcore_barrier()
            for g in range(ng):
                pltpu.sync_copy(af_hbm.at[pl.ds(ebase + g * 128, 128)], vals_v)
                pltpu.sync_copy(vals_v, acc_sh.at[idx_v.at[g]], add=True)
            plsc.subcore_barrier()
            pltpu.sync_copy(acc_sh.at[pl.ds(s * rpt, rpt)],
                            out_hbm.at[pl.ds((q * _NC + c) * r16 + s * rpt, rpt)])
            if q + 1 < npass:
                plsc.subcore_barrier()

    return k4(af128, src, dst, zrows)


def _k5_body(a_ref, om_ref, out_ref):
    p = 8
    rpi = 32
    bi = a_ref.shape[0] // rpi
    n = rpi * 16
    ib = pl.program_id(0)
    rr = lax.broadcasted_iota(jnp.int32, (rpi, 128), 0)
    ll = lax.broadcasted_iota(jnp.int32, (rpi, 128), 1)
    jj = rr * 16 + ll // p
    gg = lax.broadcasted_iota(jnp.int32, (16, 128), 0)
    l16 = lax.broadcasted_iota(jnp.int32, (16, 128), 1)
    eexp = (l16 // p == gg).astype(_F32)
    pp = lax.broadcasted_iota(jnp.int32, (p, 128), 0)
    l8 = lax.broadcasted_iota(jnp.int32, (p, 128), 1)
    b8 = (l8 % p == pp).astype(_F32)
    for k in range(bi):
        i = ib * bi + k
        x = a_ref[pl.ds(k * rpi, rpi), :]
        x = jnp.where(jj == i, x - 10000.0, x)
        mxr = jnp.max(x, axis=0, keepdims=True)
        for w in (64, 32, 16, 8):
            mxr = jnp.maximum(mxr[:, :w], mxr[:, w:2 * w])
        mxb = lax.dot_general(mxr, b8, (((1,), (0,)), ((), ())),
                              preferred_element_type=_F32, precision=_HI)
        ex = jnp.exp(x - mxb)
        smr = jnp.sum(ex, axis=0, keepdims=True)
        for w in (64, 32, 16, 8):
            smr = smr[:, :w] + smr[:, w:2 * w]
        smb = lax.dot_general(1.0 / smr, b8, (((1,), (0,)), ((), ())),
                              preferred_element_type=_F32, precision=_HI)
        omk = lax.dot_general(om_ref[pl.ds(k * rpi, rpi), :], eexp,
                              (((1,), (0,)), ((), ())),
                              preferred_element_type=_F32, precision=_HI)
        out_ref[pl.ds(k * rpi, rpi), :] = ex * smb * omk


def _k5(a128, om16):
    nr = om16.shape[0]
    bi = 8
    rpi = 32
    return pl.pallas_call(
        _k5_body,
        grid=(nr // (bi * rpi),),
        in_specs=[pl.BlockSpec((bi * rpi, 128), lambda i: (i, 0)),
                  pl.BlockSpec((bi * rpi, 16), lambda i: (i, 0))],
        out_specs=pl.BlockSpec((bi * rpi, 128), lambda i: (i, 0)),
        out_shape=jax.ShapeDtypeStruct((nr, 128), _F32),
    )(a128, om16)


def kernel(obj_feats, union_feats, pair_idxs, Ws, bs, Wo, bo, Ww, bw):
    n, d = obj_feats.shape
    e = union_feats.shape[0]
    p = Ww.shape[0]
    src = pair_idxs[:, 0].astype(jnp.int32)
    dst = pair_idxs[:, 1].astype(jnp.int32)
    hs, ho, om = _k1(obj_feats, Ws, bs[None, :], Wo, bo[None, :])
    hsg, hog = _k2(hs, ho, src, dst)
    flat3 = (src * n + dst).reshape(e // 1024, 1, 1024)
    af128 = _k3(hsg, hog, union_feats, Ww, bw[None, :], flat3)
    zrows = jnp.zeros((n * n // 16 // (2 * _NC) // _NS, 128), _F32)
    a128 = _k4(af128, src, dst, zrows, n)
    om16 = om.reshape(n * n // 16, 16)
    return _k5(a128, om16).reshape(n, n, p)


# speedup vs baseline: 1.4390x; 1.4390x over previous
import functools

import jax
import jax.numpy as jnp
from jax import lax
from jax.experimental import pallas as pl
from jax.experimental.pallas import tpu as pltpu
from jax.experimental.pallas import tpu_sc as plsc

_F32 = jnp.float32
_HI = lax.Precision.HIGHEST

_NC = 2
_NS = 16


def _dotT(a, b):
    return lax.dot_general(a, b, (((1,), (1,)), ((), ())),
                           preferred_element_type=_F32, precision=_HI)


def _k1_body(obj_ref, ws_ref, bs_ref, wo_ref, bo_ref, hs_ref, ho_ref, om_ref):
    obj = obj_ref[...]
    n = obj.shape[0]
    hs_ref[...] = _dotT(obj, ws_ref[...]) + bs_ref[...]
    ho_ref[...] = _dotT(obj, wo_ref[...]) + bo_ref[...]
    g = _dotT(obj, obj)
    s1 = jnp.sum(obj * obj, axis=1)
    sq = s1[:, None] + s1[None, :] - 2.0 * g
    om = jnp.where(sq < 0.25, 4.0, jnp.where(sq < 1.0, 1.0 / sq, 0.0))
    ii = lax.broadcasted_iota(jnp.int32, (n, n), 0)
    jj = lax.broadcasted_iota(jnp.int32, (n, n), 1)
    om_ref[...] = jnp.where(ii == jj, 0.0, om)


def _k1(obj, Ws, bs2, Wo, bo2):
    n, d = obj.shape
    return pl.pallas_call(
        _k1_body,
        out_shape=[jax.ShapeDtypeStruct((n, d), _F32),
                   jax.ShapeDtypeStruct((n, d), _F32),
                   jax.ShapeDtypeStruct((n, n), _F32)],
    )(obj, Ws, bs2, Wo, bo2)


def _k2(hs, ho, src, dst):
    n, d = hs.shape
    e = src.shape[0]
    nw = _NC * _NS
    e_per_w = e // nw
    chunk = 64
    nchunk = e_per_w // chunk
    mesh = plsc.VectorSubcoreMesh(core_axis_name="c", subcore_axis_name="s",
                                  num_cores=_NC, num_subcores=_NS)

    @functools.partial(
        pl.kernel,
        out_type=[jax.ShapeDtypeStruct((e, d), _F32),
                  jax.ShapeDtypeStruct((e, d), _F32)],
        mesh=mesh,
        scratch_types=[
            pltpu.VMEM((chunk,), jnp.int32),
            pltpu.VMEM((chunk, d), _F32),
            pltpu.SemaphoreType.DMA,
        ],
    )
    def k2(hs_hbm, ho_hbm, src_hbm, dst_hbm, hsg_hbm, hog_hbm,
           idx_v, rows_v, sem):
        wid = lax.axis_index("s") * _NC + lax.axis_index("c")
        for tab, idxarr, out in ((hs_hbm, src_hbm, hsg_hbm),
                                 (ho_hbm, dst_hbm, hog_hbm)):
            for c in range(nchunk):
                base = wid * e_per_w + c * chunk
                pltpu.sync_copy(idxarr.at[pl.ds(base, chunk)], idx_v)
                pltpu.async_copy(tab.at[idx_v], rows_v, sem).wait()
                pltpu.sync_copy(rows_v, out.at[pl.ds(base, chunk)])

    return k2(hs, ho, src, dst)


def _k3_body(hsg_ref, hog_ref, un_ref, ww_ref, bw_ref, flat_ref, out_ref):
    m = hsg_ref[...] * hog_ref[...] * un_ref[...]
    be, p = out_ref.shape[0], ww_ref.shape[0]
    af = _dotT(m, ww_ref[...]) + bw_ref[...]
    flat = flat_ref[0, 0, :]
    af16 = jnp.broadcast_to(af[:, None, :], (be, 16, p)).reshape(be, 16 * p)
    lane = lax.broadcasted_iota(jnp.int32, (be, 16 * p), 1)
    sel = (lane // p) == (flat % 16)[:, None]
    out_ref[...] = jnp.where(sel, af16, 0.0)


def _k3(hsg, hog, union, Ww, bw2, flat3):
    e, d = hsg.shape
    p = Ww.shape[0]
    be = 1024
    grid = (e // be,)
    row_spec = pl.BlockSpec((be, d), lambda i: (i, 0))
    return pl.pallas_call(
        _k3_body,
        grid=grid,
        in_specs=[row_spec, row_spec, row_spec,
                  pl.BlockSpec((p, d), lambda i: (0, 0)),
                  pl.BlockSpec((1, p), lambda i: (0, 0)),
                  pl.BlockSpec((1, 1, be), lambda i: (i, 0, 0))],
        out_specs=pl.BlockSpec((be, 16 * p), lambda i: (i, 0)),
        out_shape=jax.ShapeDtypeStruct((e, 16 * p), _F32),
    )(hsg, hog, union, Ww, bw2, flat3)


def _k4(af128, src, dst, zrows, n):
    e = af128.shape[0]
    npass = 2
    region = n * n // (npass * _NC)
    r16 = region // 16
    ept = e // _NS
    rpt = r16 // _NS
    ng = ept // 128
    mesh = plsc.VectorSubcoreMesh(core_axis_name="c", subcore_axis_name="s",
                                  num_cores=_NC, num_subcores=_NS)

    @functools.partial(
        pl.kernel,
        out_type=jax.ShapeDtypeStruct((n * n // 16, 128), _F32),
        mesh=mesh,
        scratch_types=[
            pltpu.VMEM((128, 128), _F32),
            pltpu.VMEM((ept,), jnp.int32),
            pltpu.VMEM((ept,), jnp.int32),
            pltpu.VMEM((ng, 128), jnp.int32),
            pltpu.VMEM_SHARED((r16 + 1, 128), _F32),
            pltpu.SemaphoreType.DMA,
        ],
    )
    def k4(af_hbm, src_hbm, dst_hbm, z_hbm, out_hbm,
           vals_v, src_v, dst_v, idx_v, acc_sh, sem):
        c = lax.axis_index("c")
        s = lax.axis_index("s")
        ebase = s * ept
        pltpu.sync_copy(src_hbm.at[pl.ds(ebase, ept)], src_v)
        pltpu.sync_copy(dst_hbm.at[pl.ds(ebase, ept)], dst_v)
        for q in range(npass):
            lo = (q * _NC + c) * region
            pltpu.sync_copy(z_hbm, acc_sh.at[pl.ds(s * rpt, rpt)])
            for k in range(ept // 16):
                s16 = src_v[pl.ds(k * 16, 16)]
                d16 = dst_v[pl.ds(k * 16, 16)]
                flat = s16 * n + d16
                inh = (flat >= lo) & (flat < lo + region)
                row = lax.shift_right_arithmetic(flat - lo, 4)
                idx_v[k // 8, pl.ds((k % 8) * 16, 16)] = jnp.where(inh, row, r16)
            plsc.subcore_barrier()
            for g in range(ng):
                pltpu.sync_copy(af_hbm.at[pl.ds(ebase + g * 128, 128)], vals_v)
                pltpu.sync_copy(vals_v, acc_sh.at[idx_v.at[g]], add=True)
            plsc.subcore_barrier()
            pltpu.sync_copy(acc_sh.at[pl.ds(s * rpt, rpt)],
                            out_hbm.at[pl.ds((q * _NC + c) * r16 + s * rpt, rpt)])
            if q + 1 < npass:
                plsc.subcore_barrier()

    return k4(af128, src, dst, zrows)


def _k5_body(a_ref, om_ref, out_ref):
    p = 8
    rpi = 32
    bi = a_ref.shape[0] // rpi
    n = rpi * 16
    ib = pl.program_id(0)
    br = bi * rpi
    rr = lax.broadcasted_iota(jnp.int32, (br, 128), 0)
    ll = lax.broadcasted_iota(jnp.int32, (br, 128), 1)
    jj = (rr % rpi) * 16 + ll // p
    ii = ib * bi + rr // rpi
    gg = lax.broadcasted_iota(jnp.int32, (16, 128), 0)
    l16 = lax.broadcasted_iota(jnp.int32, (16, 128), 1)
    eexp = (l16 // p == gg).astype(_F32)
    x = jnp.where(jj == ii, a_ref[...] - 10000.0, a_ref[...])
    mx = jnp.max(x.reshape(bi, rpi, 128), axis=1)
    for w in (64, 32, 16, 8):
        mx = jnp.maximum(mx[:, :w], mx[:, w:2 * w])
    for _ in range(4):
        mx = jnp.concatenate([mx, mx], axis=1)
    mxb = jnp.broadcast_to(mx[:, None, :], (bi, rpi, 128)).reshape(br, 128)
    ex = jnp.exp(x - mxb)
    sm = jnp.sum(ex.reshape(bi, rpi, 128), axis=1)
    for w in (64, 32, 16, 8):
        sm = sm[:, :w] + sm[:, w:2 * w]
    sm = 1.0 / sm
    for _ in range(4):
        sm = jnp.concatenate([sm, sm], axis=1)
    smb = jnp.broadcast_to(sm[:, None, :], (bi, rpi, 128)).reshape(br, 128)
    omk = lax.dot_general(om_ref[...], eexp, (((1,), (0,)), ((), ())),
                          preferred_element_type=_F32, precision=_HI)
    out_ref[...] = ex * smb * omk


def _k5(a128, om16):
    nr = om16.shape[0]
    bi = 8
    rpi = 32
    return pl.pallas_call(
        _k5_body,
        grid=(nr // (bi * rpi),),
        in_specs=[pl.BlockSpec((bi * rpi, 128), lambda i: (i, 0)),
                  pl.BlockSpec((bi * rpi, 16), lambda i: (i, 0))],
        out_specs=pl.BlockSpec((bi * rpi, 128), lambda i: (i, 0)),
        out_shape=jax.ShapeDtypeStruct((nr, 128), _F32),
    )(a128, om16)


def kernel(obj_feats, union_feats, pair_idxs, Ws, bs, Wo, bo, Ww, bw):
    n, d = obj_feats.shape
    e = union_feats.shape[0]
    p = Ww.shape[0]
    src = pair_idxs[:, 0].astype(jnp.int32)
    dst = pair_idxs[:, 1].astype(jnp.int32)
    hs, ho, om = _k1(obj_feats, Ws, bs[None, :], Wo, bo[None, :])
    hsg, hog = _k2(hs, ho, src, dst)
    flat3 = (src * n + dst).reshape(e // 1024, 1, 1024)
    af128 = _k3(hsg, hog, union_feats, Ww, bw[None, :], flat3)
    zrows = jnp.zeros((n * n // 16 // (2 * _NC) // _NS, 128), _F32)
    a128 = _k4(af128, src, dst, zrows, n)
    om16 = om.reshape(n * n // 16, 16)
    return _k5(a128, om16).reshape(n, n, p)


# speedup vs baseline: 1.4885x; 1.0344x over previous
import functools

import jax
import jax.numpy as jnp
from jax import lax
from jax.experimental import pallas as pl
from jax.experimental.pallas import tpu as pltpu
from jax.experimental.pallas import tpu_sc as plsc

_F32 = jnp.float32
_HI = lax.Precision.HIGHEST

_NC = 2
_NS = 16


def _dotT(a, b):
    return lax.dot_general(a, b, (((1,), (1,)), ((), ())),
                           preferred_element_type=_F32, precision=_HI)


def _k1_body(obj_ref, ws_ref, bs_ref, wo_ref, bo_ref, hs_ref, ho_ref, om_ref):
    obj = obj_ref[...]
    n = obj.shape[0]
    hs_ref[...] = _dotT(obj, ws_ref[...]) + bs_ref[...]
    ho_ref[...] = _dotT(obj, wo_ref[...]) + bo_ref[...]
    g = _dotT(obj, obj)
    s1 = jnp.sum(obj * obj, axis=1)
    sq = s1[:, None] + s1[None, :] - 2.0 * g
    om = jnp.where(sq < 0.25, 4.0, jnp.where(sq < 1.0, 1.0 / sq, 0.0))
    ii = lax.broadcasted_iota(jnp.int32, (n, n), 0)
    jj = lax.broadcasted_iota(jnp.int32, (n, n), 1)
    om_ref[...] = jnp.where(ii == jj, 0.0, om)


def _k1(obj, Ws, bs2, Wo, bo2):
    n, d = obj.shape
    return pl.pallas_call(
        _k1_body,
        out_shape=[jax.ShapeDtypeStruct((n, d), _F32),
                   jax.ShapeDtypeStruct((n, d), _F32),
                   jax.ShapeDtypeStruct((n, n), _F32)],
    )(obj, Ws, bs2, Wo, bo2)


def _k2(hs, ho, src, dst):
    n, d = hs.shape
    e = src.shape[0]
    nw = _NC * _NS
    e_per_w = e // nw
    chunk = 64
    nchunk = e_per_w // chunk
    mesh = plsc.VectorSubcoreMesh(core_axis_name="c", subcore_axis_name="s",
                                  num_cores=_NC, num_subcores=_NS)

    @functools.partial(
        pl.kernel,
        out_type=[jax.ShapeDtypeStruct((e, d), _F32),
                  jax.ShapeDtypeStruct((e, d), _F32)],
        mesh=mesh,
        scratch_types=[
            pltpu.VMEM((e_per_w,), jnp.int32),
            pltpu.VMEM((e_per_w,), jnp.int32),
            pltpu.VMEM((chunk, d), _F32),
            pltpu.VMEM((chunk, d), _F32),
            pltpu.SemaphoreType.DMA,
            pltpu.SemaphoreType.DMA,
            pltpu.SemaphoreType.DMA,
            pltpu.SemaphoreType.DMA,
        ],
    )
    def k2(hs_hbm, ho_hbm, src_hbm, dst_hbm, hsg_hbm, hog_hbm,
           idxs_v, idxd_v, rows0_v, rows1_v, gs0, gs1, ws0, ws1):
        wid = lax.axis_index("s") * _NC + lax.axis_index("c")
        base0 = wid * e_per_w
        pltpu.sync_copy(src_hbm.at[pl.ds(base0, e_per_w)], idxs_v)
        pltpu.sync_copy(dst_hbm.at[pl.ds(base0, e_per_w)], idxd_v)
        rows = (rows0_v, rows1_v)
        gsem = (gs0, gs1)
        wsem = (ws0, ws1)
        steps = [(hs_hbm, idxs_v, hsg_hbm, c) for c in range(nchunk)] + \
                [(ho_hbm, idxd_v, hog_hbm, c) for c in range(nchunk)]
        gd = [None, None]
        wd = [None, None]
        for t, (tab, idxr, out, c) in enumerate(steps):
            b = t % 2
            if wd[b] is not None:
                wd[b].wait()
            gd[b] = pltpu.async_copy(
                tab.at[idxr.at[pl.ds(c * chunk, chunk)]], rows[b], gsem[b])
            if t >= 1:
                pb = (t - 1) % 2
                tabp, idxp, outp, cp = steps[t - 1]
                gd[pb].wait()
                wd[pb] = pltpu.async_copy(
                    rows[pb], outp.at[pl.ds(base0 + cp * chunk, chunk)],
                    wsem[pb])
        lb = (len(steps) - 1) % 2
        tabl, idxl, outl, cl = steps[-1]
        gd[lb].wait()
        wd[lb] = pltpu.async_copy(
            rows[lb], outl.at[pl.ds(base0 + cl * chunk, chunk)], wsem[lb])
        wd[0].wait()
        wd[1].wait()

    return k2(hs, ho, src, dst)


def _k3_body(hsg_ref, hog_ref, un_ref, ww_ref, bw_ref, flat_ref, out_ref):
    m = hsg_ref[...] * hog_ref[...] * un_ref[...]
    be, p = out_ref.shape[0], ww_ref.shape[0]
    af = _dotT(m, ww_ref[...]) + bw_ref[...]
    flat = flat_ref[0, 0, :]
    af16 = jnp.broadcast_to(af[:, None, :], (be, 16, p)).reshape(be, 16 * p)
    lane = lax.broadcasted_iota(jnp.int32, (be, 16 * p), 1)
    sel = (lane // p) == (flat % 16)[:, None]
    out_ref[...] = jnp.where(sel, af16, 0.0)


def _k3(hsg, hog, union, Ww, bw2, flat3):
    e, d = hsg.shape
    p = Ww.shape[0]
    be = 1024
    grid = (e // be,)
    row_spec = pl.BlockSpec((be, d), lambda i: (i, 0))
    return pl.pallas_call(
        _k3_body,
        grid=grid,
        in_specs=[row_spec, row_spec, row_spec,
                  pl.BlockSpec((p, d), lambda i: (0, 0)),
                  pl.BlockSpec((1, p), lambda i: (0, 0)),
                  pl.BlockSpec((1, 1, be), lambda i: (i, 0, 0))],
        out_specs=pl.BlockSpec((be, 16 * p), lambda i: (i, 0)),
        out_shape=jax.ShapeDtypeStruct((e, 16 * p), _F32),
    )(hsg, hog, union, Ww, bw2, flat3)


def _k4(af128, src, dst, zrows, n):
    e = af128.shape[0]
    npass = 2
    region = n * n // (npass * _NC)
    r16 = region // 16
    ept = e // _NS
    rpt = r16 // _NS
    ng = ept // 128
    mesh = plsc.VectorSubcoreMesh(core_axis_name="c", subcore_axis_name="s",
                                  num_cores=_NC, num_subcores=_NS)

    @functools.partial(
        pl.kernel,
        out_type=jax.ShapeDtypeStruct((n * n // 16, 128), _F32),
        mesh=mesh,
        scratch_types=[
            pltpu.VMEM((128, 128), _F32),
            pltpu.VMEM((128, 128), _F32),
            pltpu.VMEM((ept,), jnp.int32),
            pltpu.VMEM((ept,), jnp.int32),
            pltpu.VMEM((ng, 128), jnp.int32),
            pltpu.VMEM_SHARED((r16 + 1, 128), _F32),
            pltpu.SemaphoreType.DMA,
            pltpu.SemaphoreType.DMA,
            pltpu.SemaphoreType.DMA,
            pltpu.SemaphoreType.DMA,
        ],
    )
    def k4(af_hbm, src_hbm, dst_hbm, z_hbm, out_hbm,
           vals0_v, vals1_v, src_v, dst_v, idx_v, acc_sh, ls0, ls1, ss0, ss1):
        c = lax.axis_index("c")
        s = lax.axis_index("s")
        ebase = s * ept
        pltpu.sync_copy(src_hbm.at[pl.ds(ebase, ept)], src_v)
        pltpu.sync_copy(dst_hbm.at[pl.ds(ebase, ept)], dst_v)
        vals = (vals0_v, vals1_v)
        lsem = (ls0, ls1)
        ssem = (ss0, ss1)
        for q in range(npass):
            lo = (q * _NC + c) * region
            pltpu.sync_copy(z_hbm, acc_sh.at[pl.ds(s * rpt, rpt)])
            for k in range(ept // 16):
                s16 = src_v[pl.ds(k * 16, 16)]
                d16 = dst_v[pl.ds(k * 16, 16)]
                flat = s16 * n + d16
                inh = (flat >= lo) & (flat < lo + region)
                row = lax.shift_right_arithmetic(flat - lo, 4)
                idx_v[k // 8, pl.ds((k % 8) * 16, 16)] = jnp.where(inh, row, r16)
            plsc.subcore_barrier()
            ld = [None, None]
            sd = [None, None]
            for g in range(ng):
                b = g % 2
                if sd[b] is not None:
                    sd[b].wait()
                ld[b] = pltpu.async_copy(
                    af_hbm.at[pl.ds(ebase + g * 128, 128)], vals[b], lsem[b])
                if g >= 1:
                    pb = (g - 1) % 2
                    ld[pb].wait()
                    sd[pb] = pltpu.async_copy(
                        vals[pb], acc_sh.at[idx_v.at[g - 1]], ssem[pb],
                        add=True)
            lb = (ng - 1) % 2
            ld[lb].wait()
            sd[lb] = pltpu.async_copy(
                vals[lb], acc_sh.at[idx_v.at[ng - 1]], ssem[lb], add=True)
            sd[0].wait()
            sd[1].wait()
            plsc.subcore_barrier()
            pltpu.sync_copy(acc_sh.at[pl.ds(s * rpt, rpt)],
                            out_hbm.at[pl.ds((q * _NC + c) * r16 + s * rpt, rpt)])
            if q + 1 < npass:
                plsc.subcore_barrier()

    return k4(af128, src, dst, zrows)


def _k5_body(a_ref, om_ref, out_ref):
    p = 8
    rpi = 32
    bi = a_ref.shape[0] // rpi
    n = rpi * 16
    ib = pl.program_id(0)
    br = bi * rpi
    rr = lax.broadcasted_iota(jnp.int32, (br, 128), 0)
    ll = lax.broadcasted_iota(jnp.int32, (br, 128), 1)
    jj = (rr % rpi) * 16 + ll // p
    ii = ib * bi + rr // rpi
    gg = lax.broadcasted_iota(jnp.int32, (16, 128), 0)
    l16 = lax.broadcasted_iota(jnp.int32, (16, 128), 1)
    eexp = (l16 // p == gg).astype(_F32)
    x = jnp.where(jj == ii, a_ref[...] - 10000.0, a_ref[...])
    mx = jnp.max(x.reshape(bi, rpi, 128), axis=1)
    for w in (64, 32, 16, 8):
        mx = jnp.maximum(mx[:, :w], mx[:, w:2 * w])
    for _ in range(4):
        mx = jnp.concatenate([mx, mx], axis=1)
    mxb = jnp.broadcast_to(mx[:, None, :], (bi, rpi, 128)).reshape(br, 128)
    ex = jnp.exp(x - mxb)
    sm = jnp.sum(ex.reshape(bi, rpi, 128), axis=1)
    for w in (64, 32, 16, 8):
        sm = sm[:, :w] + sm[:, w:2 * w]
    sm = 1.0 / sm
    for _ in range(4):
        sm = jnp.concatenate([sm, sm], axis=1)
    smb = jnp.broadcast_to(sm[:, None, :], (bi, rpi, 128)).reshape(br, 128)
    omk = lax.dot_general(om_ref[...], eexp, (((1,), (0,)), ((), ())),
                          preferred_element_type=_F32, precision=_HI)
    out_ref[...] = ex * smb * omk


def _k5(a128, om16):
    nr = om16.shape[0]
    bi = 8
    rpi = 32
    return pl.pallas_call(
        _k5_body,
        grid=(nr // (bi * rpi),),
        in_specs=[pl.BlockSpec((bi * rpi, 128), lambda i: (i, 0)),
                  pl.BlockSpec((bi * rpi, 16), lambda i: (i, 0))],
        out_specs=pl.BlockSpec((bi * rpi, 128), lambda i: (i, 0)),
        out_shape=jax.ShapeDtypeStruct((nr, 128), _F32),
    )(a128, om16)


def kernel(obj_feats, union_feats, pair_idxs, Ws, bs, Wo, bo, Ww, bw):
    n, d = obj_feats.shape
    e = union_feats.shape[0]
    p = Ww.shape[0]
    src = pair_idxs[:, 0].astype(jnp.int32)
    dst = pair_idxs[:, 1].astype(jnp.int32)
    hs, ho, om = _k1(obj_feats, Ws, bs[None, :], Wo, bo[None, :])
    hsg, hog = _k2(hs, ho, src, dst)
    flat3 = (src * n + dst).reshape(e // 1024, 1, 1024)
    af128 = _k3(hsg, hog, union_feats, Ww, bw[None, :], flat3)
    zrows = jnp.zeros((n * n // 16 // (2 * _NC) // _NS, 128), _F32)
    a128 = _k4(af128, src, dst, zrows, n)
    om16 = om.reshape(n * n // 16, 16)
    return _k5(a128, om16).reshape(n, n, p)


# speedup vs baseline: 1.6371x; 1.0998x over previous
import functools

import jax
import jax.numpy as jnp
from jax import lax
from jax.experimental import pallas as pl
from jax.experimental.pallas import tpu as pltpu
from jax.experimental.pallas import tpu_sc as plsc

_F32 = jnp.float32
_HI = lax.Precision.HIGHEST

_NC = 2
_NS = 16


def _dotT(a, b):
    return lax.dot_general(a, b, (((1,), (1,)), ((), ())),
                           preferred_element_type=_F32, precision=_HI)


def _k1_body(obj_ref, ws_ref, bs_ref, wo_ref, bo_ref, hs_ref, ho_ref, om_ref):
    obj = obj_ref[...]
    n = obj.shape[0]
    hs_ref[...] = _dotT(obj, ws_ref[...]) + bs_ref[...]
    ho_ref[...] = _dotT(obj, wo_ref[...]) + bo_ref[...]
    g = _dotT(obj, obj)
    s1 = jnp.sum(obj * obj, axis=1)
    sq = s1[:, None] + s1[None, :] - 2.0 * g
    om = jnp.where(sq < 0.25, 4.0, jnp.where(sq < 1.0, 1.0 / sq, 0.0))
    ii = lax.broadcasted_iota(jnp.int32, (n, n), 0)
    jj = lax.broadcasted_iota(jnp.int32, (n, n), 1)
    om_ref[...] = jnp.where(ii == jj, 0.0, om)


def _k1(obj, Ws, bs2, Wo, bo2):
    n, d = obj.shape
    return pl.pallas_call(
        _k1_body,
        out_shape=[jax.ShapeDtypeStruct((n, d), _F32),
                   jax.ShapeDtypeStruct((n, d), _F32),
                   jax.ShapeDtypeStruct((n, n), _F32)],
    )(obj, Ws, bs2, Wo, bo2)


def _k2(hs, ho, src, dst):
    n, d = hs.shape
    e = src.shape[0]
    nw = _NC * _NS
    e_per_w = e // nw
    chunk = 64
    nchunk = e_per_w // chunk
    mesh = plsc.VectorSubcoreMesh(core_axis_name="c", subcore_axis_name="s",
                                  num_cores=_NC, num_subcores=_NS)

    @functools.partial(
        pl.kernel,
        out_type=[jax.ShapeDtypeStruct((e, d), _F32),
                  jax.ShapeDtypeStruct((e, d), _F32)],
        mesh=mesh,
        scratch_types=[
            pltpu.VMEM((e_per_w,), jnp.int32),
            pltpu.VMEM((e_per_w,), jnp.int32),
            pltpu.VMEM((chunk, d), _F32),
            pltpu.VMEM((chunk, d), _F32),
            pltpu.SemaphoreType.DMA,
            pltpu.SemaphoreType.DMA,
            pltpu.SemaphoreType.DMA,
            pltpu.SemaphoreType.DMA,
        ],
    )
    def k2(hs_hbm, ho_hbm, src_hbm, dst_hbm, hsg_hbm, hog_hbm,
           idxs_v, idxd_v, rows0_v, rows1_v, gs0, gs1, ws0, ws1):
        wid = lax.axis_index("s") * _NC + lax.axis_index("c")
        base0 = wid * e_per_w
        pltpu.sync_copy(src_hbm.at[pl.ds(base0, e_per_w)], idxs_v)
        pltpu.sync_copy(dst_hbm.at[pl.ds(base0, e_per_w)], idxd_v)
        rows = (rows0_v, rows1_v)
        gsem = (gs0, gs1)
        wsem = (ws0, ws1)
        steps = [(hs_hbm, idxs_v, hsg_hbm, c) for c in range(nchunk)] + \
                [(ho_hbm, idxd_v, hog_hbm, c) for c in range(nchunk)]
        gd = [None, None]
        wd = [None, None]
        for t, (tab, idxr, out, c) in enumerate(steps):
            b = t % 2
            if wd[b] is not None:
                wd[b].wait()
            gd[b] = pltpu.async_copy(
                tab.at[idxr.at[pl.ds(c * chunk, chunk)]], rows[b], gsem[b])
            if t >= 1:
                pb = (t - 1) % 2
                tabp, idxp, outp, cp = steps[t - 1]
                gd[pb].wait()
                wd[pb] = pltpu.async_copy(
                    rows[pb], outp.at[pl.ds(base0 + cp * chunk, chunk)],
                    wsem[pb])
        lb = (len(steps) - 1) % 2
        tabl, idxl, outl, cl = steps[-1]
        gd[lb].wait()
        wd[lb] = pltpu.async_copy(
            rows[lb], outl.at[pl.ds(base0 + cl * chunk, chunk)], wsem[lb])
        wd[0].wait()
        wd[1].wait()

    return k2(hs, ho, src, dst)


def _k3_body(hsg_ref, hog_ref, un_ref, ww_ref, bw_ref, flat_ref, out_ref):
    m = hsg_ref[...] * hog_ref[...] * un_ref[...]
    be, p = out_ref.shape[0], ww_ref.shape[0]
    af = _dotT(m, ww_ref[...]) + bw_ref[...]
    flat = flat_ref[0, 0, :]
    af16 = jnp.broadcast_to(af[:, None, :], (be, 16, p)).reshape(be, 16 * p)
    lane = lax.broadcasted_iota(jnp.int32, (be, 16 * p), 1)
    sel = (lane // p) == (flat % 16)[:, None]
    out_ref[...] = jnp.where(sel, af16, 0.0)


def _k3(hsg, hog, union, Ww, bw2, flat3):
    e, d = hsg.shape
    p = Ww.shape[0]
    be = 2048
    grid = (e // be,)
    row_spec = pl.BlockSpec((be, d), lambda i: (i, 0))
    return pl.pallas_call(
        _k3_body,
        grid=grid,
        in_specs=[row_spec, row_spec, row_spec,
                  pl.BlockSpec((p, d), lambda i: (0, 0)),
                  pl.BlockSpec((1, p), lambda i: (0, 0)),
                  pl.BlockSpec((1, 1, be), lambda i: (i, 0, 0))],
        out_specs=pl.BlockSpec((be, 16 * p), lambda i: (i, 0)),
        out_shape=jax.ShapeDtypeStruct((e, 16 * p), _F32),
    )(hsg, hog, union, Ww, bw2, flat3)


def _k4(af128, src, dst, zrows, n):
    e = af128.shape[0]
    npass = 2
    region = n * n // (npass * _NC)
    r16 = region // 16
    ept = e // _NS
    rpt = r16 // _NS
    ng = ept // 128
    mesh = plsc.VectorSubcoreMesh(core_axis_name="c", subcore_axis_name="s",
                                  num_cores=_NC, num_subcores=_NS)

    @functools.partial(
        pl.kernel,
        out_type=jax.ShapeDtypeStruct((n * n // 16, 128), _F32),
        mesh=mesh,
        scratch_types=[
            pltpu.VMEM((128, 128), _F32),
            pltpu.VMEM((128, 128), _F32),
            pltpu.VMEM((ept,), jnp.int32),
            pltpu.VMEM((ept,), jnp.int32),
            pltpu.VMEM((ng, 128), jnp.int32),
            pltpu.VMEM_SHARED((r16 + 1, 128), _F32),
            pltpu.SemaphoreType.DMA,
            pltpu.SemaphoreType.DMA,
            pltpu.SemaphoreType.DMA,
            pltpu.SemaphoreType.DMA,
        ],
    )
    def k4(af_hbm, src_hbm, dst_hbm, z_hbm, out_hbm,
           vals0_v, vals1_v, src_v, dst_v, idx_v, acc_sh, ls0, ls1, ss0, ss1):
        c = lax.axis_index("c")
        s = lax.axis_index("s")
        ebase = s * ept
        pltpu.sync_copy(src_hbm.at[pl.ds(ebase, ept)], src_v)
        pltpu.sync_copy(dst_hbm.at[pl.ds(ebase, ept)], dst_v)
        vals = (vals0_v, vals1_v)
        lsem = (ls0, ls1)
        ssem = (ss0, ss1)
        for q in range(npass):
            lo = (q * _NC + c) * region
            pltpu.sync_copy(z_hbm, acc_sh.at[pl.ds(s * rpt, rpt)])
            for k in range(ept // 16):
                s16 = src_v[pl.ds(k * 16, 16)]
                d16 = dst_v[pl.ds(k * 16, 16)]
                flat = s16 * n + d16
                inh = (flat >= lo) & (flat < lo + region)
                row = lax.shift_right_arithmetic(flat - lo, 4)
                idx_v[k // 8, pl.ds((k % 8) * 16, 16)] = jnp.where(inh, row, r16)
            plsc.subcore_barrier()
            ld = [None, None]
            sd = [None, None]
            for g in range(ng):
                b = g % 2
                if sd[b] is not None:
                    sd[b].wait()
                ld[b] = pltpu.async_copy(
                    af_hbm.at[pl.ds(ebase + g * 128, 128)], vals[b], lsem[b])
                if g >= 1:
                    pb = (g - 1) % 2
                    ld[pb].wait()
                    sd[pb] = pltpu.async_copy(
                        vals[pb], acc_sh.at[idx_v.at[g - 1]], ssem[pb],
                        add=True)
            lb = (ng - 1) % 2
            ld[lb].wait()
            sd[lb] = pltpu.async_copy(
                vals[lb], acc_sh.at[idx_v.at[ng - 1]], ssem[lb], add=True)
            sd[0].wait()
            sd[1].wait()
            plsc.subcore_barrier()
            pltpu.sync_copy(acc_sh.at[pl.ds(s * rpt, rpt)],
                            out_hbm.at[pl.ds((q * _NC + c) * r16 + s * rpt, rpt)])
            if q + 1 < npass:
                plsc.subcore_barrier()

    return k4(af128, src, dst, zrows)


def _k5_body(a_ref, om_ref, out_ref):
    p = 8
    rpi = 32
    bi = a_ref.shape[0] // rpi
    n = rpi * 16
    ib = pl.program_id(0)
    br = bi * rpi
    rr = lax.broadcasted_iota(jnp.int32, (br, 128), 0)
    ll = lax.broadcasted_iota(jnp.int32, (br, 128), 1)
    jj = (rr % rpi) * 16 + ll // p
    ii = ib * bi + rr // rpi
    gg = lax.broadcasted_iota(jnp.int32, (16, 128), 0)
    l16 = lax.broadcasted_iota(jnp.int32, (16, 128), 1)
    eexp = (l16 // p == gg).astype(_F32)
    x = jnp.where(jj == ii, a_ref[...] - 10000.0, a_ref[...])
    mx = jnp.max(x.reshape(bi, rpi, 128), axis=1)
    for w in (64, 32, 16, 8):
        mx = jnp.maximum(mx[:, :w], mx[:, w:2 * w])
    for _ in range(4):
        mx = jnp.concatenate([mx, mx], axis=1)
    mxb = jnp.broadcast_to(mx[:, None, :], (bi, rpi, 128)).reshape(br, 128)
    ex = jnp.exp(x - mxb)
    sm = jnp.sum(ex.reshape(bi, rpi, 128), axis=1)
    for w in (64, 32, 16, 8):
        sm = sm[:, :w] + sm[:, w:2 * w]
    sm = 1.0 / sm
    for _ in range(4):
        sm = jnp.concatenate([sm, sm], axis=1)
    smb = jnp.broadcast_to(sm[:, None, :], (bi, rpi, 128)).reshape(br, 128)
    omk = lax.dot_general(om_ref[...], eexp, (((1,), (0,)), ((), ())),
                          preferred_element_type=_F32, precision=_HI)
    out_ref[...] = ex * smb * omk


def _k5(a128, om16):
    nr = om16.shape[0]
    bi = 16
    rpi = 32
    return pl.pallas_call(
        _k5_body,
        grid=(nr // (bi * rpi),),
        in_specs=[pl.BlockSpec((bi * rpi, 128), lambda i: (i, 0)),
                  pl.BlockSpec((bi * rpi, 16), lambda i: (i, 0))],
        out_specs=pl.BlockSpec((bi * rpi, 128), lambda i: (i, 0)),
        out_shape=jax.ShapeDtypeStruct((nr, 128), _F32),
    )(a128, om16)


def kernel(obj_feats, union_feats, pair_idxs, Ws, bs, Wo, bo, Ww, bw):
    n, d = obj_feats.shape
    e = union_feats.shape[0]
    p = Ww.shape[0]
    src = pair_idxs[:, 0].astype(jnp.int32)
    dst = pair_idxs[:, 1].astype(jnp.int32)
    hs, ho, om = _k1(obj_feats, Ws, bs[None, :], Wo, bo[None, :])
    hsg, hog = _k2(hs, ho, src, dst)
    flat3 = (src * n + dst).reshape(e // 2048, 1, 2048)
    af128 = _k3(hsg, hog, union_feats, Ww, bw[None, :], flat3)
    zrows = jnp.zeros((n * n // 16 // (2 * _NC) // _NS, 128), _F32)
    a128 = _k4(af128, src, dst, zrows, n)
    om16 = om.reshape(n * n // 16, 16)
    return _k5(a128, om16).reshape(n, n, p)


# speedup vs baseline: 1.7201x; 1.0507x over previous
import functools

import jax
import jax.numpy as jnp
from jax import lax
from jax.experimental import pallas as pl
from jax.experimental.pallas import tpu as pltpu
from jax.experimental.pallas import tpu_sc as plsc

_F32 = jnp.float32
_HI = lax.Precision.HIGHEST

_NC = 2
_NS = 16


def _dotT(a, b):
    return lax.dot_general(a, b, (((1,), (1,)), ((), ())),
                           preferred_element_type=_F32, precision=_HI)


def _k1_body(obj_ref, ws_ref, bs_ref, wo_ref, bo_ref, hs_ref, ho_ref, om_ref):
    obj = obj_ref[...]
    n = obj.shape[0]
    hs_ref[...] = _dotT(obj, ws_ref[...]) + bs_ref[...]
    ho_ref[...] = _dotT(obj, wo_ref[...]) + bo_ref[...]
    g = _dotT(obj, obj)
    s1 = jnp.sum(obj * obj, axis=1)
    sq = s1[:, None] + s1[None, :] - 2.0 * g
    om = jnp.where(sq < 0.25, 4.0, jnp.where(sq < 1.0, 1.0 / sq, 0.0))
    ii = lax.broadcasted_iota(jnp.int32, (n, n), 0)
    jj = lax.broadcasted_iota(jnp.int32, (n, n), 1)
    om_ref[...] = jnp.where(ii == jj, 0.0, om)


def _k1(obj, Ws, bs2, Wo, bo2):
    n, d = obj.shape
    return pl.pallas_call(
        _k1_body,
        out_shape=[jax.ShapeDtypeStruct((n, d), _F32),
                   jax.ShapeDtypeStruct((n, d), _F32),
                   jax.ShapeDtypeStruct((n, n), _F32)],
    )(obj, Ws, bs2, Wo, bo2)


def _k2(hs, ho, src, dst):
    n, d = hs.shape
    e = src.shape[0]
    nw = _NC * _NS
    e_per_w = e // nw
    chunk = 64
    nchunk = e_per_w // chunk
    mesh = plsc.VectorSubcoreMesh(core_axis_name="c", subcore_axis_name="s",
                                  num_cores=_NC, num_subcores=_NS)

    @functools.partial(
        pl.kernel,
        out_type=[jax.ShapeDtypeStruct((e, d), _F32),
                  jax.ShapeDtypeStruct((e, d), _F32)],
        mesh=mesh,
        scratch_types=[
            pltpu.VMEM((e_per_w,), jnp.int32),
            pltpu.VMEM((e_per_w,), jnp.int32),
            pltpu.VMEM((chunk, d), _F32),
            pltpu.VMEM((chunk, d), _F32),
            pltpu.SemaphoreType.DMA,
            pltpu.SemaphoreType.DMA,
            pltpu.SemaphoreType.DMA,
            pltpu.SemaphoreType.DMA,
        ],
    )
    def k2(hs_hbm, ho_hbm, src_hbm, dst_hbm, hsg_hbm, hog_hbm,
           idxs_v, idxd_v, rows0_v, rows1_v, gs0, gs1, ws0, ws1):
        wid = lax.axis_index("s") * _NC + lax.axis_index("c")
        base0 = wid * e_per_w
        pltpu.sync_copy(src_hbm.at[pl.ds(base0, e_per_w)], idxs_v)
        pltpu.sync_copy(dst_hbm.at[pl.ds(base0, e_per_w)], idxd_v)
        rows = (rows0_v, rows1_v)
        gsem = (gs0, gs1)
        wsem = (ws0, ws1)
        steps = [(hs_hbm, idxs_v, hsg_hbm, c) for c in range(nchunk)] + \
                [(ho_hbm, idxd_v, hog_hbm, c) for c in range(nchunk)]
        gd = [None, None]
        wd = [None, None]
        for t, (tab, idxr, out, c) in enumerate(steps):
            b = t % 2
            if wd[b] is not None:
                wd[b].wait()
            gd[b] = pltpu.async_copy(
                tab.at[idxr.at[pl.ds(c * chunk, chunk)]], rows[b], gsem[b])
            if t >= 1:
                pb = (t - 1) % 2
                tabp, idxp, outp, cp = steps[t - 1]
                gd[pb].wait()
                wd[pb] = pltpu.async_copy(
                    rows[pb], outp.at[pl.ds(base0 + cp * chunk, chunk)],
                    wsem[pb])
        lb = (len(steps) - 1) % 2
        tabl, idxl, outl, cl = steps[-1]
        gd[lb].wait()
        wd[lb] = pltpu.async_copy(
            rows[lb], outl.at[pl.ds(base0 + cl * chunk, chunk)], wsem[lb])
        wd[0].wait()
        wd[1].wait()

    return k2(hs, ho, src, dst)


def _k3_body(hsg_ref, hog_ref, un_ref, ww_ref, bw_ref, flat_ref, out_ref):
    m = hsg_ref[...] * hog_ref[...] * un_ref[...]
    be, p = out_ref.shape[0], ww_ref.shape[0]
    af = _dotT(m, ww_ref[...]) + bw_ref[...]
    flat = flat_ref[0, 0, :]
    af16 = jnp.broadcast_to(af[:, None, :], (be, 16, p)).reshape(be, 16 * p)
    lane = lax.broadcasted_iota(jnp.int32, (be, 16 * p), 1)
    sel = (lane // p) == (flat % 16)[:, None]
    out_ref[...] = jnp.where(sel, af16, 0.0)


def _k3(hsg, hog, union, Ww, bw2, flat3):
    e, d = hsg.shape
    p = Ww.shape[0]
    be = 2048
    grid = (e // be,)
    row_spec = pl.BlockSpec((be, d), lambda i: (i, 0))
    return pl.pallas_call(
        _k3_body,
        grid=grid,
        in_specs=[row_spec, row_spec, row_spec,
                  pl.BlockSpec((p, d), lambda i: (0, 0)),
                  pl.BlockSpec((1, p), lambda i: (0, 0)),
                  pl.BlockSpec((1, 1, be), lambda i: (i, 0, 0))],
        out_specs=pl.BlockSpec((be, 16 * p), lambda i: (i, 0)),
        out_shape=jax.ShapeDtypeStruct((e, 16 * p), _F32),
    )(hsg, hog, union, Ww, bw2, flat3)


def _k4(af128, src, dst, zrows, n):
    e = af128.shape[0]
    npass = 2
    region = n * n // (npass * _NC)
    r16 = region // 16
    ept = e // _NS
    rpt = r16 // _NS
    ng = ept // 128
    mesh = plsc.VectorSubcoreMesh(core_axis_name="c", subcore_axis_name="s",
                                  num_cores=_NC, num_subcores=_NS)

    @functools.partial(
        pl.kernel,
        out_type=jax.ShapeDtypeStruct((n * n // 16, 128), _F32),
        mesh=mesh,
        scratch_types=[
            pltpu.VMEM((128, 128), _F32),
            pltpu.VMEM((128, 128), _F32),
            pltpu.VMEM((ept,), jnp.int32),
            pltpu.VMEM((ept,), jnp.int32),
            pltpu.VMEM((ng, 128), jnp.int32),
            pltpu.VMEM_SHARED((r16 + 1, 128), _F32),
            pltpu.SemaphoreType.DMA,
            pltpu.SemaphoreType.DMA,
            pltpu.SemaphoreType.DMA,
            pltpu.SemaphoreType.DMA,
        ],
    )
    def k4(af_hbm, src_hbm, dst_hbm, z_hbm, out_hbm,
           vals0_v, vals1_v, src_v, dst_v, idx_v, acc_sh, ls0, ls1, ss0, ss1):
        c = lax.axis_index("c")
        s = lax.axis_index("s")
        ebase = s * ept
        pltpu.sync_copy(src_hbm.at[pl.ds(ebase, ept)], src_v)
        pltpu.sync_copy(dst_hbm.at[pl.ds(ebase, ept)], dst_v)
        vals = (vals0_v, vals1_v)
        lsem = (ls0, ls1)
        ssem = (ss0, ss1)
        for q in range(npass):
            lo = (q * _NC + c) * region
            pltpu.sync_copy(z_hbm, acc_sh.at[pl.ds(s * rpt, rpt)])
            for k in range(ept // 16):
                s16 = src_v[pl.ds(k * 16, 16)]
                d16 = dst_v[pl.ds(k * 16, 16)]
                flat = s16 * n + d16
                inh = (flat >= lo) & (flat < lo + region)
                row = lax.shift_right_arithmetic(flat - lo, 4)
                idx_v[k // 8, pl.ds((k % 8) * 16, 16)] = jnp.where(inh, row, r16)
            plsc.subcore_barrier()
            ld = [None, None]
            sd = [None, None]
            for g in range(ng):
                b = g % 2
                if sd[b] is not None:
                    sd[b].wait()
                ld[b] = pltpu.async_copy(
                    af_hbm.at[pl.ds(ebase + g * 128, 128)], vals[b], lsem[b])
                if g >= 1:
                    pb = (g - 1) % 2
                    ld[pb].wait()
                    sd[pb] = pltpu.async_copy(
                        vals[pb], acc_sh.at[idx_v.at[g - 1]], ssem[pb],
                        add=True)
            lb = (ng - 1) % 2
            ld[lb].wait()
            sd[lb] = pltpu.async_copy(
                vals[lb], acc_sh.at[idx_v.at[ng - 1]], ssem[lb], add=True)
            sd[0].wait()
            sd[1].wait()
            plsc.subcore_barrier()
            pltpu.sync_copy(acc_sh.at[pl.ds(s * rpt, rpt)],
                            out_hbm.at[pl.ds((q * _NC + c) * r16 + s * rpt, rpt)])
            if q + 1 < npass:
                plsc.subcore_barrier()

    return k4(af128, src, dst, zrows)


def _k5_body(a_ref, om_ref, out_ref):
    p = 8
    rpi = 32
    bi = a_ref.shape[0] // rpi
    n = rpi * 16
    ib = pl.program_id(0)
    br = bi * rpi
    rr = lax.broadcasted_iota(jnp.int32, (br, 128), 0)
    ll = lax.broadcasted_iota(jnp.int32, (br, 128), 1)
    jj = (rr % rpi) * 16 + ll // p
    ii = ib * bi + rr // rpi
    gg = lax.broadcasted_iota(jnp.int32, (16, 128), 0)
    l16 = lax.broadcasted_iota(jnp.int32, (16, 128), 1)
    eexp = (l16 // p == gg).astype(_F32)
    x = jnp.where(jj == ii, a_ref[...] - 10000.0, a_ref[...])
    mx = jnp.max(x.reshape(bi, rpi, 128), axis=1)
    for w in (64, 32, 16, 8):
        mx = jnp.maximum(mx[:, :w], mx[:, w:2 * w])
    for _ in range(4):
        mx = jnp.concatenate([mx, mx], axis=1)
    mxb = jnp.broadcast_to(mx[:, None, :], (bi, rpi, 128)).reshape(br, 128)
    ex = jnp.exp(x - mxb)
    sm = jnp.sum(ex.reshape(bi, rpi, 128), axis=1)
    for w in (64, 32, 16, 8):
        sm = sm[:, :w] + sm[:, w:2 * w]
    sm = 1.0 / sm
    for _ in range(4):
        sm = jnp.concatenate([sm, sm], axis=1)
    smb = jnp.broadcast_to(sm[:, None, :], (bi, rpi, 128)).reshape(br, 128)
    omk = lax.dot_general(om_ref[...], eexp, (((1,), (0,)), ((), ())),
                          preferred_element_type=_F32, precision=_HI)
    out_ref[...] = ex * smb * omk


def _k5(a128, om16):
    nr = om16.shape[0]
    bi = 32
    rpi = 32
    return pl.pallas_call(
        _k5_body,
        grid=(nr // (bi * rpi),),
        in_specs=[pl.BlockSpec((bi * rpi, 128), lambda i: (i, 0)),
                  pl.BlockSpec((bi * rpi, 16), lambda i: (i, 0))],
        out_specs=pl.BlockSpec((bi * rpi, 128), lambda i: (i, 0)),
        out_shape=jax.ShapeDtypeStruct((nr, 128), _F32),
    )(a128, om16)


def kernel(obj_feats, union_feats, pair_idxs, Ws, bs, Wo, bo, Ww, bw):
    n, d = obj_feats.shape
    e = union_feats.shape[0]
    p = Ww.shape[0]
    src = pair_idxs[:, 0].astype(jnp.int32)
    dst = pair_idxs[:, 1].astype(jnp.int32)
    hs, ho, om = _k1(obj_feats, Ws, bs[None, :], Wo, bo[None, :])
    hsg, hog = _k2(hs, ho, src, dst)
    flat3 = (src * n + dst).reshape(e // 2048, 1, 2048)
    af128 = _k3(hsg, hog, union_feats, Ww, bw[None, :], flat3)
    zrows = jnp.zeros((n * n // 16 // (2 * _NC) // _NS, 128), _F32)
    a128 = _k4(af128, src, dst, zrows, n)
    om16 = om.reshape(n * n // 16, 16)
    return _k5(a128, om16).reshape(n, n, p)


# speedup vs baseline: 1.7576x; 1.0218x over previous
import functools

import jax
import jax.numpy as jnp
from jax import lax
from jax.experimental import pallas as pl
from jax.experimental.pallas import tpu as pltpu
from jax.experimental.pallas import tpu_sc as plsc

_F32 = jnp.float32
_HI = lax.Precision.HIGHEST

_NC = 2
_NS = 16


def _dotT(a, b):
    return lax.dot_general(a, b, (((1,), (1,)), ((), ())),
                           preferred_element_type=_F32, precision=_HI)


def _k1_body(obj_ref, ws_ref, bs_ref, wo_ref, bo_ref, hs_ref, ho_ref, om_ref):
    obj = obj_ref[...]
    n = obj.shape[0]
    hs_ref[...] = _dotT(obj, ws_ref[...]) + bs_ref[...]
    ho_ref[...] = _dotT(obj, wo_ref[...]) + bo_ref[...]
    g = _dotT(obj, obj)
    s1 = jnp.sum(obj * obj, axis=1)
    sq = s1[:, None] + s1[None, :] - 2.0 * g
    om = jnp.where(sq < 0.25, 4.0, jnp.where(sq < 1.0, 1.0 / sq, 0.0))
    ii = lax.broadcasted_iota(jnp.int32, (n, n), 0)
    jj = lax.broadcasted_iota(jnp.int32, (n, n), 1)
    om_ref[...] = jnp.where(ii == jj, 0.0, om)


def _k1(obj, Ws, bs2, Wo, bo2):
    n, d = obj.shape
    return pl.pallas_call(
        _k1_body,
        out_shape=[jax.ShapeDtypeStruct((n, d), _F32),
                   jax.ShapeDtypeStruct((n, d), _F32),
                   jax.ShapeDtypeStruct((n, n), _F32)],
    )(obj, Ws, bs2, Wo, bo2)


def _k2(hs, ho, src, dst):
    n, d = hs.shape
    e = src.shape[0]
    nw = _NC * _NS
    e_per_w = e // nw
    chunk = 64
    nchunk = e_per_w // chunk
    mesh = plsc.VectorSubcoreMesh(core_axis_name="c", subcore_axis_name="s",
                                  num_cores=_NC, num_subcores=_NS)

    @functools.partial(
        pl.kernel,
        out_type=[jax.ShapeDtypeStruct((e, d), _F32),
                  jax.ShapeDtypeStruct((e, d), _F32)],
        mesh=mesh,
        scratch_types=[
            pltpu.VMEM((e_per_w,), jnp.int32),
            pltpu.VMEM((e_per_w,), jnp.int32),
            pltpu.VMEM((chunk, d), _F32),
            pltpu.VMEM((chunk, d), _F32),
            pltpu.SemaphoreType.DMA,
            pltpu.SemaphoreType.DMA,
            pltpu.SemaphoreType.DMA,
            pltpu.SemaphoreType.DMA,
        ],
    )
    def k2(hs_hbm, ho_hbm, src_hbm, dst_hbm, hsg_hbm, hog_hbm,
           idxs_v, idxd_v, rows0_v, rows1_v, gs0, gs1, ws0, ws1):
        wid = lax.axis_index("s") * _NC + lax.axis_index("c")
        base0 = wid * e_per_w
        pltpu.sync_copy(src_hbm.at[pl.ds(base0, e_per_w)], idxs_v)
        pltpu.sync_copy(dst_hbm.at[pl.ds(base0, e_per_w)], idxd_v)
        rows = (rows0_v, rows1_v)
        gsem = (gs0, gs1)
        wsem = (ws0, ws1)
        steps = [(hs_hbm, idxs_v, hsg_hbm, c) for c in range(nchunk)] + \
                [(ho_hbm, idxd_v, hog_hbm, c) for c in range(nchunk)]
        gd = [None, None]
        wd = [None, None]
        for t, (tab, idxr, out, c) in enumerate(steps):
            b = t % 2
            if wd[b] is not None:
                wd[b].wait()
            gd[b] = pltpu.async_copy(
                tab.at[idxr.at[pl.ds(c * chunk, chunk)]], rows[b], gsem[b])
            if t >= 1:
                pb = (t - 1) % 2
                tabp, idxp, outp, cp = steps[t - 1]
                gd[pb].wait()
                wd[pb] = pltpu.async_copy(
                    rows[pb], outp.at[pl.ds(base0 + cp * chunk, chunk)],
                    wsem[pb])
        lb = (len(steps) - 1) % 2
        tabl, idxl, outl, cl = steps[-1]
        gd[lb].wait()
        wd[lb] = pltpu.async_copy(
            rows[lb], outl.at[pl.ds(base0 + cl * chunk, chunk)], wsem[lb])
        wd[0].wait()
        wd[1].wait()

    return k2(hs, ho, src, dst)


def _k3_body(hsg_ref, hog_ref, un_ref, ww_ref, bw_ref, flat_ref, out_ref):
    m = hsg_ref[...] * hog_ref[...] * un_ref[...]
    be, p = out_ref.shape[0], ww_ref.shape[0]
    af = _dotT(m, ww_ref[...]) + bw_ref[...]
    flat = flat_ref[0, 0, :]
    af16 = jnp.broadcast_to(af[:, None, :], (be, 16, p)).reshape(be, 16 * p)
    lane = lax.broadcasted_iota(jnp.int32, (be, 16 * p), 1)
    sel = (lane // p) == (flat % 16)[:, None]
    out_ref[...] = jnp.where(sel, af16, 0.0)


def _k3(hsg, hog, union, Ww, bw2, flat3):
    e, d = hsg.shape
    p = Ww.shape[0]
    be = 2048
    grid = (e // be,)
    row_spec = pl.BlockSpec((be, d), lambda i: (i, 0))
    return pl.pallas_call(
        _k3_body,
        grid=grid,
        in_specs=[row_spec, row_spec, row_spec,
                  pl.BlockSpec((p, d), lambda i: (0, 0)),
                  pl.BlockSpec((1, p), lambda i: (0, 0)),
                  pl.BlockSpec((1, 1, be), lambda i: (i, 0, 0))],
        out_specs=pl.BlockSpec((be, 16 * p), lambda i: (i, 0)),
        out_shape=jax.ShapeDtypeStruct((e, 16 * p), _F32),
    )(hsg, hog, union, Ww, bw2, flat3)


def _k4(af128, src, dst, zrows, n):
    e = af128.shape[0]
    npass = 2
    region = n * n // (npass * _NC)
    r16 = region // 16
    ept = e // _NS
    rpt = r16 // _NS
    ng = ept // 128
    mesh = plsc.VectorSubcoreMesh(core_axis_name="c", subcore_axis_name="s",
                                  num_cores=_NC, num_subcores=_NS)

    @functools.partial(
        pl.kernel,
        out_type=jax.ShapeDtypeStruct((n * n // 16, 128), _F32),
        mesh=mesh,
        scratch_types=[
            pltpu.VMEM((128, 128), _F32),
            pltpu.VMEM((128, 128), _F32),
            pltpu.VMEM((ept,), jnp.int32),
            pltpu.VMEM((ept,), jnp.int32),
            pltpu.VMEM((ng, 128), jnp.int32),
            pltpu.VMEM_SHARED((r16 + 1, 128), _F32),
            pltpu.SemaphoreType.DMA,
            pltpu.SemaphoreType.DMA,
            pltpu.SemaphoreType.DMA,
            pltpu.SemaphoreType.DMA,
        ],
    )
    def k4(af_hbm, src_hbm, dst_hbm, z_hbm, out_hbm,
           vals0_v, vals1_v, src_v, dst_v, idx_v, acc_sh, ls0, ls1, ss0, ss1):
        c = lax.axis_index("c")
        s = lax.axis_index("s")
        ebase = s * ept
        pltpu.sync_copy(src_hbm.at[pl.ds(ebase, ept)], src_v)
        pltpu.sync_copy(dst_hbm.at[pl.ds(ebase, ept)], dst_v)
        vals = (vals0_v, vals1_v)
        lsem = (ls0, ls1)
        ssem = (ss0, ss1)
        for q in range(npass):
            lo = (q * _NC + c) * region
            pltpu.sync_copy(z_hbm, acc_sh.at[pl.ds(s * rpt, rpt)])
            for k in range(ept // 16):
                s16 = src_v[pl.ds(k * 16, 16)]
                d16 = dst_v[pl.ds(k * 16, 16)]
                flat = s16 * n + d16
                inh = (flat >= lo) & (flat < lo + region)
                row = lax.shift_right_arithmetic(flat - lo, 4)
                idx_v[k // 8, pl.ds((k % 8) * 16, 16)] = jnp.where(inh, row, r16)
            plsc.subcore_barrier()
            ld = [None, None]
            sd = [None, None]
            for g in range(ng):
                b = g % 2
                if sd[b] is not None:
                    sd[b].wait()
                ld[b] = pltpu.async_copy(
                    af_hbm.at[pl.ds(ebase + g * 128, 128)], vals[b], lsem[b])
                if g >= 1:
                    pb = (g - 1) % 2
                    ld[pb].wait()
                    sd[pb] = pltpu.async_copy(
                        vals[pb], acc_sh.at[idx_v.at[g - 1]], ssem[pb],
                        add=True)
            lb = (ng - 1) % 2
            ld[lb].wait()
            sd[lb] = pltpu.async_copy(
                vals[lb], acc_sh.at[idx_v.at[ng - 1]], ssem[lb], add=True)
            sd[0].wait()
            sd[1].wait()
            plsc.subcore_barrier()
            pltpu.sync_copy(acc_sh.at[pl.ds(s * rpt, rpt)],
                            out_hbm.at[pl.ds((q * _NC + c) * r16 + s * rpt, rpt)])
            if q + 1 < npass:
                plsc.subcore_barrier()

    return k4(af128, src, dst, zrows)


def _k5_body(a_ref, om_ref, out_ref):
    p = 8
    rpi = 32
    bi = a_ref.shape[0] // rpi
    n = rpi * 16
    ib = pl.program_id(0)
    br = bi * rpi
    rr = lax.broadcasted_iota(jnp.int32, (br, 128), 0)
    ll = lax.broadcasted_iota(jnp.int32, (br, 128), 1)
    jj = (rr % rpi) * 16 + ll // p
    ii = ib * bi + rr // rpi
    gg = lax.broadcasted_iota(jnp.int32, (16, 128), 0)
    l16 = lax.broadcasted_iota(jnp.int32, (16, 128), 1)
    eexp = (l16 // p == gg).astype(_F32)
    x = jnp.where(jj == ii, a_ref[...] - 10000.0, a_ref[...])
    mx = jnp.max(x.reshape(bi, rpi, 128), axis=1)
    for w in (64, 32, 16, 8):
        mx = jnp.maximum(mx[:, :w], mx[:, w:2 * w])
    for _ in range(4):
        mx = jnp.concatenate([mx, mx], axis=1)
    mxb = jnp.broadcast_to(mx[:, None, :], (bi, rpi, 128)).reshape(br, 128)
    ex = jnp.exp(x - mxb)
    sm = jnp.sum(ex.reshape(bi, rpi, 128), axis=1)
    for w in (64, 32, 16, 8):
        sm = sm[:, :w] + sm[:, w:2 * w]
    sm = 1.0 / sm
    for _ in range(4):
        sm = jnp.concatenate([sm, sm], axis=1)
    smb = jnp.broadcast_to(sm[:, None, :], (bi, rpi, 128)).reshape(br, 128)
    omk = lax.dot_general(om_ref[...], eexp, (((1,), (0,)), ((), ())),
                          preferred_element_type=_F32, precision=_HI)
    out_ref[...] = ex * smb * omk


def _k5(a128, om16):
    nr = om16.shape[0]
    bi = 64
    rpi = 32
    return pl.pallas_call(
        _k5_body,
        grid=(nr // (bi * rpi),),
        in_specs=[pl.BlockSpec((bi * rpi, 128), lambda i: (i, 0)),
                  pl.BlockSpec((bi * rpi, 16), lambda i: (i, 0))],
        out_specs=pl.BlockSpec((bi * rpi, 128), lambda i: (i, 0)),
        out_shape=jax.ShapeDtypeStruct((nr, 128), _F32),
    )(a128, om16)


def kernel(obj_feats, union_feats, pair_idxs, Ws, bs, Wo, bo, Ww, bw):
    n, d = obj_feats.shape
    e = union_feats.shape[0]
    p = Ww.shape[0]
    src = pair_idxs[:, 0].astype(jnp.int32)
    dst = pair_idxs[:, 1].astype(jnp.int32)
    hs, ho, om = _k1(obj_feats, Ws, bs[None, :], Wo, bo[None, :])
    hsg, hog = _k2(hs, ho, src, dst)
    flat3 = (src * n + dst).reshape(e // 2048, 1, 2048)
    af128 = _k3(hsg, hog, union_feats, Ww, bw[None, :], flat3)
    zrows = jnp.zeros((n * n // 16 // (2 * _NC) // _NS, 128), _F32)
    a128 = _k4(af128, src, dst, zrows, n)
    om16 = om.reshape(n * n // 16, 16)
    return _k5(a128, om16).reshape(n, n, p)
